# baseline (speedup 1.0000x reference)
import functools

import jax
import jax.numpy as jnp
from jax import lax
from jax.experimental import pallas as pl
from jax.experimental.pallas import tpu as pltpu
from jax.experimental.pallas import tpu_sc as plsc

B = 16384
D = 64
NC = 2
NS = 16
NW = NC * NS
BPW = B // NW
CH = 128
NCH = BPW // CH

_f32 = jnp.float32


def _gather_body(uidx, iidx, cidx, bidx, ut, it, ct, bt,
                 uo, io, co, bo,
                 uix, iix, cix, bix, ur, ir, cr, br, sem):
    wid = lax.axis_index("s") * NC + lax.axis_index("c")
    base = wid * BPW
    pltpu.sync_copy(uidx.at[wid], uix)
    pltpu.sync_copy(iidx.at[wid], iix)
    pltpu.sync_copy(cidx.at[wid], cix)
    pltpu.sync_copy(bidx.at[wid], bix)
    copies = []
    for j in range(NCH):
        sl = pl.ds(j * CH, CH)
        copies.append(pltpu.async_copy(ut.at[uix.at[j]], ur.at[sl], sem))
        copies.append(pltpu.async_copy(it.at[iix.at[j]], ir.at[sl], sem))
        copies.append(pltpu.async_copy(ct.at[cix.at[j]], cr.at[sl], sem))
        copies.append(pltpu.async_copy(bt.at[bix.at[j]], br.at[sl], sem))
    for c in copies:
        c.wait()
    out_sl = pl.ds(base, BPW)
    pltpu.sync_copy(ur, uo.at[out_sl])
    pltpu.sync_copy(ir, io.at[out_sl])
    pltpu.sync_copy(cr, co.at[out_sl])
    pltpu.sync_copy(br, bo.at[out_sl])


_gather = pl.kernel(
    _gather_body,
    out_type=(
        jax.ShapeDtypeStruct((B, D), _f32),
        jax.ShapeDtypeStruct((B, D), _f32),
        jax.ShapeDtypeStruct((B, D // 2), _f32),
        jax.ShapeDtypeStruct((B, D // 2), _f32),
    ),
    mesh=plsc.VectorSubcoreMesh(core_axis_name="c", subcore_axis_name="s",
                                num_cores=NC, num_subcores=NS),
    scratch_types=[
        pltpu.VMEM((NCH, CH), jnp.int32),
        pltpu.VMEM((NCH, CH), jnp.int32),
        pltpu.VMEM((NCH, CH), jnp.int32),
        pltpu.VMEM((NCH, CH), jnp.int32),
        pltpu.VMEM((BPW, D), _f32),
        pltpu.VMEM((BPW, D), _f32),
        pltpu.VMEM((BPW, D // 2), _f32),
        pltpu.VMEM((BPW, D // 2), _f32),
        pltpu.SemaphoreType.DMA,
    ],
    compiler_params=pltpu.CompilerParams(use_tc_tiling_on_sc=False),
)


def _bn(h, gamma, beta):
    mean = jnp.mean(h, axis=0, keepdims=True)
    var = jnp.mean((h - mean) ** 2, axis=0, keepdims=True)
    return (h - mean) / jnp.sqrt(var + 1e-5) * gamma + beta


def _mlp_body(u_ref, i_ref, c_ref, b_ref, w1u_ref, w1i_ref, w1c_ref, w1b_ref,
              b1_ref, g1_ref, be1_ref, w2_ref, b2_ref, g2_ref, be2_ref,
              w3_ref, b3_ref, o_ref):
    h = (jnp.dot(u_ref[...], w1u_ref[...], preferred_element_type=_f32)
         + jnp.dot(i_ref[...], w1i_ref[...], preferred_element_type=_f32)
         + jnp.dot(c_ref[...], w1c_ref[...], preferred_element_type=_f32)
         + jnp.dot(b_ref[...], w1b_ref[...], preferred_element_type=_f32)
         + b1_ref[...])
    h = jnp.maximum(_bn(h, g1_ref[...], be1_ref[...]), 0.0)
    h = jnp.dot(h, w2_ref[...], preferred_element_type=_f32) + b2_ref[...]
    h = jnp.maximum(_bn(h, g2_ref[...], be2_ref[...]), 0.0)
    out = jnp.dot(h, w3_ref[...], preferred_element_type=_f32) + b3_ref[...]
    o_ref[...] = jax.nn.sigmoid(out)


_mlp = pl.pallas_call(
    _mlp_body,
    out_shape=jax.ShapeDtypeStruct((B, 1), _f32),
)


def kernel(user_idx, item_idx, cat_idx, brand_idx, user_table, item_table,
           cat_table, brand_table, W1, b1, g1, be1, W2, b2, g2, be2, W3, b3):
    uidx = user_idx.astype(jnp.int32).reshape(NW, NCH, CH)
    iidx = item_idx.astype(jnp.int32).reshape(NW, NCH, CH)
    cidx = cat_idx.astype(jnp.int32).reshape(NW, NCH, CH)
    bidx = brand_idx.astype(jnp.int32).reshape(NW, NCH, CH)
    u, i, c, b = _gather(uidx, iidx, cidx, bidx,
                         user_table, item_table, cat_table, brand_table)
    w1u = W1[:D]
    w1i = W1[D:2 * D]
    w1c = W1[2 * D:2 * D + D // 2]
    w1b = W1[2 * D + D // 2:]
    b1r = b1.reshape(1, -1)
    g1r = g1.reshape(1, -1)
    be1r = be1.reshape(1, -1)
    b2r = b2.reshape(1, -1)
    g2r = g2.reshape(1, -1)
    be2r = be2.reshape(1, -1)
    b3r = b3.reshape(1, -1)
    out = _mlp(u, i, c, b, w1u, w1i, w1c, w1b, b1r, g1r, be1r,
               W2, b2r, g2r, be2r, W3, b3r)
    return jnp.squeeze(out, axis=-1)


# speedup vs baseline: 1.5161x; 1.5161x over previous
import jax
import jax.numpy as jnp
from jax import lax
from jax.experimental import pallas as pl
from jax.experimental.pallas import tpu as pltpu
from jax.experimental.pallas import tpu_sc as plsc

B = 16384
D = 64
H = D // 2
NC = 2
NS = 16
NW = NC * NS
BPW = B // NW

_f32 = jnp.float32


CHK = 128
NQ = BPW // CHK


def _issue_rows(idx_ref, q, table, rows, sem):
    def issue(g, _):
        v = idx_ref[pl.ds(q * CHK + g * 16, 16)]
        for l in range(16):
            r = v[l]
            pltpu.async_copy(table.at[pl.ds(r, 1)],
                             rows.at[pl.ds(g * 16 + l, 1)], sem)
        return 0

    lax.fori_loop(0, CHK // 16, issue, 0)


def _gather_body(uidx, iidx, cidx, bidx, ut, it, ct, bt,
                 uo, io, co, bo,
                 uix, iix, cix, bix, ur, ir, cr, br, sem):
    wid = lax.axis_index("s") * NC + lax.axis_index("c")
    base = wid * BPW
    pltpu.sync_copy(uidx.at[wid], uix)
    pltpu.sync_copy(iidx.at[wid], iix)
    pltpu.sync_copy(cidx.at[wid], cix)
    pltpu.sync_copy(bidx.at[wid], bix)
    for q in range(NQ):
        sl = pl.ds(base + q * CHK, CHK)
        _issue_rows(uix, q, ut, ur, sem)
        _issue_rows(iix, q, it, ir, sem)
        _issue_rows(cix, q, ct, cr, sem)
        _issue_rows(bix, q, bt, br, sem)
        pltpu.make_async_copy(ut.at[pl.ds(0, CHK)], ur, sem).wait()
        pltpu.make_async_copy(it.at[pl.ds(0, CHK)], ir, sem).wait()
        pltpu.make_async_copy(ct.at[pl.ds(0, CHK)], cr, sem).wait()
        pltpu.make_async_copy(bt.at[pl.ds(0, CHK)], br, sem).wait()
        pltpu.sync_copy(ur, uo.at[sl])
        pltpu.sync_copy(ir, io.at[sl])
        pltpu.sync_copy(cr, co.at[sl])
        pltpu.sync_copy(br, bo.at[sl])


_gather = pl.kernel(
    _gather_body,
    out_type=(
        jax.ShapeDtypeStruct((B, D), _f32),
        jax.ShapeDtypeStruct((B, D), _f32),
        jax.ShapeDtypeStruct((B, H), _f32),
        jax.ShapeDtypeStruct((B, H), _f32),
    ),
    mesh=plsc.VectorSubcoreMesh(core_axis_name="c", subcore_axis_name="s",
                                num_cores=NC, num_subcores=NS),
    scratch_types=[
        pltpu.VMEM((BPW,), jnp.int32),
        pltpu.VMEM((BPW,), jnp.int32),
        pltpu.VMEM((BPW,), jnp.int32),
        pltpu.VMEM((BPW,), jnp.int32),
        pltpu.VMEM((CHK, D), _f32),
        pltpu.VMEM((CHK, D), _f32),
        pltpu.VMEM((CHK, H), _f32),
        pltpu.VMEM((CHK, H), _f32),
        pltpu.SemaphoreType.DMA,
    ],
)


def _bn(h, gamma, beta):
    mean = jnp.mean(h, axis=0, keepdims=True)
    var = jnp.mean((h - mean) ** 2, axis=0, keepdims=True)
    return (h - mean) / jnp.sqrt(var + 1e-5) * gamma + beta


def _mlp_body(u_ref, i_ref, c_ref, b_ref, w1u_ref, w1i_ref, w1c_ref, w1b_ref,
              b1_ref, g1_ref, be1_ref, w2_ref, b2_ref, g2_ref, be2_ref,
              w3_ref, b3_ref, o_ref):
    h = (jnp.dot(u_ref[...], w1u_ref[...], preferred_element_type=_f32)
         + jnp.dot(i_ref[...], w1i_ref[...], preferred_element_type=_f32)
         + jnp.dot(c_ref[...], w1c_ref[...], preferred_element_type=_f32)
         + jnp.dot(b_ref[...], w1b_ref[...], preferred_element_type=_f32)
         + b1_ref[...])
    h = jnp.maximum(_bn(h, g1_ref[...], be1_ref[...]), 0.0)
    h = jnp.dot(h, w2_ref[...], preferred_element_type=_f32) + b2_ref[...]
    h = jnp.maximum(_bn(h, g2_ref[...], be2_ref[...]), 0.0)
    out = jnp.dot(h, w3_ref[...], preferred_element_type=_f32) + b3_ref[...]
    o_ref[...] = jax.nn.sigmoid(out)


_mlp = pl.pallas_call(
    _mlp_body,
    out_shape=jax.ShapeDtypeStruct((B, 1), _f32),
)


def kernel(user_idx, item_idx, cat_idx, brand_idx, user_table, item_table,
           cat_table, brand_table, W1, b1, g1, be1, W2, b2, g2, be2, W3, b3):
    uidx = user_idx.astype(jnp.int32).reshape(NW, BPW)
    iidx = item_idx.astype(jnp.int32).reshape(NW, BPW)
    cidx = cat_idx.astype(jnp.int32).reshape(NW, BPW)
    bidx = brand_idx.astype(jnp.int32).reshape(NW, BPW)
    u, i, c, b = _gather(uidx, iidx, cidx, bidx,
                         user_table, item_table, cat_table, brand_table)
    w1u = W1[:D]
    w1i = W1[D:2 * D]
    w1c = W1[2 * D:2 * D + H]
    w1b = W1[2 * D + H:]
    b1r = b1.reshape(1, -1)
    g1r = g1.reshape(1, -1)
    be1r = be1.reshape(1, -1)
    b2r = b2.reshape(1, -1)
    g2r = g2.reshape(1, -1)
    be2r = be2.reshape(1, -1)
    b3r = b3.reshape(1, -1)
    out = _mlp(u, i, c, b, w1u, w1i, w1c, w1b, b1r, g1r, be1r,
               W2, b2r, g2r, be2r, W3, b3r)
    return jnp.squeeze(out, axis=-1)


# speedup vs baseline: 1.5166x; 1.0003x over previous
import jax
import jax.numpy as jnp
from jax import lax
from jax.experimental import pallas as pl
from jax.experimental.pallas import tpu as pltpu
from jax.experimental.pallas import tpu_sc as plsc

B = 16384
D = 64
H = D // 2
NC = 2
NS = 16
NW = NC * NS
BPW = B // NW

_f32 = jnp.float32


CHK = 128
NQ = BPW // CHK


NSEM = 8


def _issue_rows(idx_ref, q, table, rows, sems):
    def issue(g, _):
        v = idx_ref[pl.ds(q * CHK + g * 16, 16)]
        for l in range(16):
            r = v[l]
            pltpu.async_copy(table.at[pl.ds(r, 1)],
                             rows.at[pl.ds(g * 16 + l, 1)], sems[l % NSEM])
        return 0

    lax.fori_loop(0, CHK // 16, issue, 0)


def _gather_body(uidx, iidx, cidx, bidx, ut, it, ct, bt,
                 uo, io, co, bo,
                 uix, iix, cix, bix, ur, ir, cr, br, *sems):
    wid = lax.axis_index("s") * NC + lax.axis_index("c")
    base = wid * BPW
    pltpu.sync_copy(uidx.at[wid], uix)
    pltpu.sync_copy(iidx.at[wid], iix)
    pltpu.sync_copy(cidx.at[wid], cix)
    pltpu.sync_copy(bidx.at[wid], bix)
    PS = CHK // NSEM
    for q in range(NQ):
        sl = pl.ds(base + q * CHK, CHK)
        _issue_rows(uix, q, ut, ur, sems)
        _issue_rows(iix, q, it, ir, sems)
        _issue_rows(cix, q, ct, cr, sems)
        _issue_rows(bix, q, bt, br, sems)
        for j in range(NSEM):
            pltpu.make_async_copy(ut.at[pl.ds(0, PS)],
                                  ur.at[pl.ds(0, PS)], sems[j]).wait()
            pltpu.make_async_copy(it.at[pl.ds(0, PS)],
                                  ir.at[pl.ds(0, PS)], sems[j]).wait()
            pltpu.make_async_copy(ct.at[pl.ds(0, PS)],
                                  cr.at[pl.ds(0, PS)], sems[j]).wait()
            pltpu.make_async_copy(bt.at[pl.ds(0, PS)],
                                  br.at[pl.ds(0, PS)], sems[j]).wait()
        pltpu.sync_copy(ur, uo.at[sl])
        pltpu.sync_copy(ir, io.at[sl])
        pltpu.sync_copy(cr, co.at[sl])
        pltpu.sync_copy(br, bo.at[sl])


_gather = pl.kernel(
    _gather_body,
    out_type=(
        jax.ShapeDtypeStruct((B, D), _f32),
        jax.ShapeDtypeStruct((B, D), _f32),
        jax.ShapeDtypeStruct((B, H), _f32),
        jax.ShapeDtypeStruct((B, H), _f32),
    ),
    mesh=plsc.VectorSubcoreMesh(core_axis_name="c", subcore_axis_name="s",
                                num_cores=NC, num_subcores=NS),
    scratch_types=[
        pltpu.VMEM((BPW,), jnp.int32),
        pltpu.VMEM((BPW,), jnp.int32),
        pltpu.VMEM((BPW,), jnp.int32),
        pltpu.VMEM((BPW,), jnp.int32),
        pltpu.VMEM((CHK, D), _f32),
        pltpu.VMEM((CHK, D), _f32),
        pltpu.VMEM((CHK, H), _f32),
        pltpu.VMEM((CHK, H), _f32),
    ] + [pltpu.SemaphoreType.DMA] * NSEM,
)


def _bn(h, gamma, beta):
    mean = jnp.mean(h, axis=0, keepdims=True)
    var = jnp.mean((h - mean) ** 2, axis=0, keepdims=True)
    return (h - mean) / jnp.sqrt(var + 1e-5) * gamma + beta


def _mlp_body(u_ref, i_ref, c_ref, b_ref, w1u_ref, w1i_ref, w1c_ref, w1b_ref,
              b1_ref, g1_ref, be1_ref, w2_ref, b2_ref, g2_ref, be2_ref,
              w3_ref, b3_ref, o_ref):
    h = (jnp.dot(u_ref[...], w1u_ref[...], preferred_element_type=_f32)
         + jnp.dot(i_ref[...], w1i_ref[...], preferred_element_type=_f32)
         + jnp.dot(c_ref[...], w1c_ref[...], preferred_element_type=_f32)
         + jnp.dot(b_ref[...], w1b_ref[...], preferred_element_type=_f32)
         + b1_ref[...])
    h = jnp.maximum(_bn(h, g1_ref[...], be1_ref[...]), 0.0)
    h = jnp.dot(h, w2_ref[...], preferred_element_type=_f32) + b2_ref[...]
    h = jnp.maximum(_bn(h, g2_ref[...], be2_ref[...]), 0.0)
    out = jnp.dot(h, w3_ref[...], preferred_element_type=_f32) + b3_ref[...]
    o_ref[...] = jax.nn.sigmoid(out)


_mlp = pl.pallas_call(
    _mlp_body,
    out_shape=jax.ShapeDtypeStruct((B, 1), _f32),
)


def kernel(user_idx, item_idx, cat_idx, brand_idx, user_table, item_table,
           cat_table, brand_table, W1, b1, g1, be1, W2, b2, g2, be2, W3, b3):
    uidx = user_idx.astype(jnp.int32).reshape(NW, BPW)
    iidx = item_idx.astype(jnp.int32).reshape(NW, BPW)
    cidx = cat_idx.astype(jnp.int32).reshape(NW, BPW)
    bidx = brand_idx.astype(jnp.int32).reshape(NW, BPW)
    u, i, c, b = _gather(uidx, iidx, cidx, bidx,
                         user_table, item_table, cat_table, brand_table)
    w1u = W1[:D]
    w1i = W1[D:2 * D]
    w1c = W1[2 * D:2 * D + H]
    w1b = W1[2 * D + H:]
    b1r = b1.reshape(1, -1)
    g1r = g1.reshape(1, -1)
    be1r = be1.reshape(1, -1)
    b2r = b2.reshape(1, -1)
    g2r = g2.reshape(1, -1)
    be2r = be2.reshape(1, -1)
    b3r = b3.reshape(1, -1)
    out = _mlp(u, i, c, b, w1u, w1i, w1c, w1b, b1r, g1r, be1r,
               W2, b2r, g2r, be2r, W3, b3r)
    return jnp.squeeze(out, axis=-1)
